# baseline (speedup 1.0000x reference)
import functools

import jax
import jax.numpy as jnp
from jax.experimental import pallas as pl

_N0, _N1, _N2, _N3, _N4 = 100000, 25000, 6250, 1562, 390
_NB = 8


def _cdiv(a, b):
    return (a + b - 1) // b


def _elu(y):
    return jnp.where(y > 0, y, jnp.exp(jnp.minimum(y, 0.0)) - 1.0)


def _msg_body(xg_ref, ps_ref, a_ref, o_ref):
    ps = ps_ref[...]
    a = a_ref[...]
    w = jnp.sum(ps * a, axis=1, keepdims=True) + 1.0
    o_ref[...] = xg_ref[...] * w


def _weight_messages(xg, pseudo, a):
    E, F = xg.shape
    B = 2048
    Ep = _cdiv(E, B) * B
    xg_p = jnp.pad(xg, ((0, Ep - E), (0, 0)))
    ps_p = jnp.pad(pseudo, ((0, Ep - E), (0, 0)))
    out = pl.pallas_call(
        _msg_body,
        grid=(Ep // B,),
        in_specs=[
            pl.BlockSpec((B, F), lambda i: (i, 0)),
            pl.BlockSpec((B, 3), lambda i: (i, 0)),
            pl.BlockSpec((1, 3), lambda i: (0, 0)),
        ],
        out_specs=pl.BlockSpec((B, F), lambda i: (i, 0)),
        out_shape=jax.ShapeDtypeStruct((Ep, F), xg.dtype),
    )(xg_p, ps_p, a.reshape(1, 3))
    return out[:E]


def _conv_body(s_ref, c_ref, x_ref, w_ref, wr_ref, b_ref, o_ref, *, elu):
    cnt = jnp.maximum(c_ref[...], 1.0)
    agg = s_ref[...] / cnt
    y = jnp.dot(agg, w_ref[...], preferred_element_type=jnp.float32)
    y = y + jnp.dot(x_ref[...], wr_ref[...], preferred_element_type=jnp.float32)
    y = y + b_ref[...]
    if elu:
        y = _elu(y)
    o_ref[...] = y


def _conv_out(s, cnt, x, W, Wr, b, elu):
    N, fi = x.shape
    fo = W.shape[1]
    B = 512
    Np = _cdiv(N, B) * B
    s_p = jnp.pad(s, ((0, Np - N), (0, 0)))
    c_p = jnp.pad(cnt.reshape(-1, 1), ((0, Np - N), (0, 0)))
    x_p = jnp.pad(x, ((0, Np - N), (0, 0)))
    out = pl.pallas_call(
        functools.partial(_conv_body, elu=elu),
        grid=(Np // B,),
        in_specs=[
            pl.BlockSpec((B, fi), lambda i: (i, 0)),
            pl.BlockSpec((B, 1), lambda i: (i, 0)),
            pl.BlockSpec((B, fi), lambda i: (i, 0)),
            pl.BlockSpec((fi, fo), lambda i: (0, 0)),
            pl.BlockSpec((fi, fo), lambda i: (0, 0)),
            pl.BlockSpec((1, fo), lambda i: (0, 0)),
        ],
        out_specs=pl.BlockSpec((B, fo), lambda i: (i, 0)),
        out_shape=jax.ShapeDtypeStruct((Np, fo), jnp.float32),
    )(s_p, c_p, x_p, W, Wr, b.reshape(1, -1))
    return out[:N]


def _resid_skip_body(h_ref, x_ref, w_ref, b_ref, o_ref):
    y = h_ref[...] + jnp.dot(x_ref[...], w_ref[...],
                             preferred_element_type=jnp.float32) + b_ref[...]
    o_ref[...] = _elu(y)


def _resid_skip(h, x, W, b):
    N, fi = x.shape
    fo = W.shape[1]
    B = 512
    Np = _cdiv(N, B) * B
    h_p = jnp.pad(h, ((0, Np - N), (0, 0)))
    x_p = jnp.pad(x, ((0, Np - N), (0, 0)))
    out = pl.pallas_call(
        _resid_skip_body,
        grid=(Np // B,),
        in_specs=[
            pl.BlockSpec((B, fo), lambda i: (i, 0)),
            pl.BlockSpec((B, fi), lambda i: (i, 0)),
            pl.BlockSpec((fi, fo), lambda i: (0, 0)),
            pl.BlockSpec((1, fo), lambda i: (0, 0)),
        ],
        out_specs=pl.BlockSpec((B, fo), lambda i: (i, 0)),
        out_shape=jax.ShapeDtypeStruct((Np, fo), jnp.float32),
    )(h_p, x_p, W, b.reshape(1, -1))
    return out[:N]


def _resid_body(h_ref, x_ref, o_ref):
    o_ref[...] = _elu(h_ref[...] + x_ref[...])


def _resid(h, x):
    N, F = x.shape
    B = 512
    Np = _cdiv(N, B) * B
    h_p = jnp.pad(h, ((0, Np - N), (0, 0)))
    x_p = jnp.pad(x, ((0, Np - N), (0, 0)))
    out = pl.pallas_call(
        _resid_body,
        grid=(Np // B,),
        in_specs=[
            pl.BlockSpec((B, F), lambda i: (i, 0)),
            pl.BlockSpec((B, F), lambda i: (i, 0)),
        ],
        out_specs=pl.BlockSpec((B, F), lambda i: (i, 0)),
        out_shape=jax.ShapeDtypeStruct((Np, F), jnp.float32),
    )(h_p, x_p)
    return out[:N]


def _head_body(s_ref, c_ref, w_ref, b_ref, o_ref):
    g = s_ref[...] / jnp.maximum(c_ref[...], 1.0)
    logits = jnp.dot(g, w_ref[...], preferred_element_type=jnp.float32) + b_ref[...]
    m = jnp.max(logits, axis=1, keepdims=True)
    lse = jnp.log(jnp.sum(jnp.exp(logits - m), axis=1, keepdims=True)) + m
    o_ref[...] = logits - lse


def _head(s, cnt, W, b):
    nb, fi = s.shape
    fo = W.shape[1]
    return pl.pallas_call(
        _head_body,
        grid=(1,),
        in_specs=[
            pl.BlockSpec((nb, fi), lambda i: (0, 0)),
            pl.BlockSpec((nb, 1), lambda i: (0, 0)),
            pl.BlockSpec((fi, fo), lambda i: (0, 0)),
            pl.BlockSpec((1, fo), lambda i: (0, 0)),
        ],
        out_specs=pl.BlockSpec((nb, fo), lambda i: (0, 0)),
        out_shape=jax.ShapeDtypeStruct((nb, fo), jnp.float32),
    )(s, cnt.reshape(-1, 1), W, b.reshape(1, -1))


def _spline_conv(p, x, ei, pseudo, n, elu):
    src, dst = ei[0], ei[1]
    xg = jnp.take(x, src, axis=0)
    msg = _weight_messages(xg, pseudo, p['a'])
    s = jax.ops.segment_sum(msg, dst, num_segments=n)
    cnt = jax.ops.segment_sum(jnp.ones((ei.shape[1],), x.dtype), dst,
                              num_segments=n)
    return _conv_out(s, cnt, x, p['W'], p['W_root'], p['b'], elu)


def _pool_max(x, cluster, n):
    out = jax.ops.segment_max(x, cluster, num_segments=n)
    return jnp.where(jnp.isfinite(out), out, 0.0)


def kernel(x, pseudo0, pseudo1, pseudo2, pseudo3, pseudo4, params,
           edge_index0, edge_index1, edge_index2, edge_index3, edge_index4,
           cluster1, cluster2, cluster3, cluster4, batch):
    P = params
    x = _spline_conv(P['conv1'], x, edge_index0, pseudo0, _N0, elu=True)
    x = _pool_max(x, cluster1, _N1)

    h = jnp.concatenate([x, jnp.ones((_N1, 1), x.dtype)], axis=1)
    h = _spline_conv(P['conv2'], h, edge_index1, pseudo1, _N1, elu=True)
    h = _spline_conv(P['conv22'], h, edge_index1, pseudo1, _N1, elu=False)
    x = _resid_skip(h, x, P['skip1']['W'], P['skip1']['b'])

    h = _spline_conv(P['conv3'], x, edge_index1, pseudo1, _N1, elu=True)
    h = _spline_conv(P['conv32'], h, edge_index1, pseudo1, _N1, elu=False)
    x = _resid(h, x)
    x = _pool_max(x, cluster2, _N2)

    h = jnp.concatenate([x, jnp.ones((_N2, 1), x.dtype)], axis=1)
    h = _spline_conv(P['conv4'], h, edge_index2, pseudo2, _N2, elu=True)
    h = _spline_conv(P['conv42'], h, edge_index2, pseudo2, _N2, elu=False)
    x = _resid_skip(h, x, P['skip2']['W'], P['skip2']['b'])
    x = _pool_max(x, cluster3, _N3)

    h = jnp.concatenate([x, jnp.ones((_N3, 1), x.dtype)], axis=1)
    h = _spline_conv(P['conv5'], h, edge_index3, pseudo3, _N3, elu=True)
    h = _spline_conv(P['conv52'], h, edge_index3, pseudo3, _N3, elu=False)
    x = _resid(h, x)
    x = _pool_max(x, cluster4, _N4)

    h = jnp.concatenate([x, jnp.ones((_N4, 1), x.dtype)], axis=1)
    h = _spline_conv(P['conv6'], h, edge_index4, pseudo4, _N4, elu=True)
    h = _spline_conv(P['conv62'], h, edge_index4, pseudo4, _N4, elu=False)
    x = _resid_skip(h, x, P['skip3']['W'], P['skip3']['b'])

    h = _spline_conv(P['conv7'], x, edge_index4, pseudo4, _N4, elu=True)
    h = _spline_conv(P['conv72'], h, edge_index4, pseudo4, _N4, elu=False)
    x = _resid(h, x)

    s = jax.ops.segment_sum(x, batch, num_segments=_NB)
    cnt = jax.ops.segment_sum(jnp.ones((_N4,), x.dtype), batch,
                              num_segments=_NB)
    return _head(s, cnt, P['fc1']['W'], P['fc1']['b'])
